# Initial kernel scaffold; baseline (speedup 1.0000x reference)
import jax
import jax.numpy as jnp
from jax.experimental import pallas as pl


def kernel(query_image, support_image, support_target, neighbors):
    raise NotImplementedError("write your pallas kernel here")


# speedup vs baseline: 3.2677x; 3.2677x over previous
import functools

import jax
import jax.numpy as jnp
from jax import lax
from jax.experimental import pallas as pl
from jax.experimental.pallas import tpu as pltpu
from jax.experimental.pallas import tpu_sc as plsc

K = 5
KPAD = 8
QBLK = 256
SBLK = 2048
IDX_BIG = 2**30


def _topk_body(q_ref, st_ref, out_ref, vals_ref, idxs_ref):
    j = pl.program_id(1)
    nj = pl.num_programs(1)

    @pl.when(j == 0)
    def _init():
        vals_ref[...] = jnp.full((QBLK, KPAD), jnp.inf, jnp.float32)
        idxs_ref[...] = jnp.full((QBLK, KPAD), IDX_BIG, jnp.int32)

    q = q_ref[...]
    st = st_ref[...]
    dots = lax.dot_general(q, st, (((1,), (0,)), ((), ())),
                           preferred_element_type=jnp.float32)
    q2 = jnp.sum(q * q, axis=1, keepdims=True)
    s2 = jnp.sum(st * st, axis=0, keepdims=True)
    d2 = (q2 + s2) - 2.0 * dots

    col = lax.broadcasted_iota(jnp.int32, (QBLK, SBLK), 1)
    base = j * SBLK
    vals = vals_ref[...]
    idxs = idxs_ref[...]
    for _ in range(K):
        m = jnp.min(d2, axis=1, keepdims=True)
        c = jnp.min(jnp.where(d2 == m, col, IDX_BIG), axis=1, keepdims=True)
        d2 = jnp.where(col == c, jnp.inf, d2)
        gi = c + base
        lt = (m < vals).astype(jnp.int32)
        p = KPAD - jnp.sum(lt, axis=1, keepdims=True)
        kcol = lax.broadcasted_iota(jnp.int32, (QBLK, KPAD), 1)
        sh_vals = jnp.concatenate([vals[:, :1], vals[:, : KPAD - 1]], axis=1)
        sh_idxs = jnp.concatenate([idxs[:, :1], idxs[:, : KPAD - 1]], axis=1)
        vals = jnp.where(kcol < p, vals, jnp.where(kcol == p, m, sh_vals))
        idxs = jnp.where(kcol < p, idxs, jnp.where(kcol == p, gi, sh_idxs))
    vals_ref[...] = vals
    idxs_ref[...] = idxs

    @pl.when(j == nj - 1)
    def _emit():
        out_ref[...] = idxs_ref[...]


def _topk_tc(q, st):
    Q, D = q.shape
    S = st.shape[1]
    return pl.pallas_call(
        _topk_body,
        grid=(Q // QBLK, S // SBLK),
        in_specs=[
            pl.BlockSpec((QBLK, D), lambda i, j: (i, 0)),
            pl.BlockSpec((D, SBLK), lambda i, j: (0, j)),
        ],
        out_specs=pl.BlockSpec((QBLK, KPAD), lambda i, j: (i, 0)),
        out_shape=jax.ShapeDtypeStruct((Q, KPAD), jnp.int32),
        scratch_shapes=[
            pltpu.VMEM((QBLK, KPAD), jnp.float32),
            pltpu.VMEM((QBLK, KPAD), jnp.int32),
        ],
        compiler_params=pltpu.CompilerParams(
            dimension_semantics=("parallel", "arbitrary"),
        ),
    )(q, st)


def _vote_sc(idx_t, support_target, nbr_vec):
    KK, Q = idx_t.shape
    S = support_target.shape[0]
    info = plsc.get_sparse_core_info()
    nw = info.num_cores * info.num_subcores
    qw = Q // nw
    ng = qw // 16

    mesh = plsc.VectorSubcoreMesh(core_axis_name="c", subcore_axis_name="s")

    @functools.partial(
        pl.kernel,
        mesh=mesh,
        out_type=jax.ShapeDtypeStruct((Q,), jnp.float32),
        scratch_types=[
            pltpu.VMEM((KK, qw), jnp.int32),
            pltpu.VMEM((S,), jnp.int32),
            pltpu.VMEM((16,), jnp.int32),
            pltpu.VMEM((qw,), jnp.float32),
        ],
        compiler_params=pltpu.CompilerParams(needs_layout_passes=False),
    )
    def vote(idx_hbm, tgt_hbm, nbr_hbm, out_hbm, idx_v, tgt_v, nbr_v, pred_v):
        wid = lax.axis_index("s") * info.num_cores + lax.axis_index("c")
        base = wid * qw
        pltpu.sync_copy(tgt_hbm, tgt_v)
        pltpu.sync_copy(idx_hbm.at[:, pl.ds(base, qw)], idx_v)
        pltpu.sync_copy(nbr_hbm, nbr_v)
        nbr = nbr_v[...]
        one = jnp.ones((16,), jnp.int32)
        zero = jnp.zeros((16,), jnp.int32)
        for g in range(ng):
            labs = []
            for jn in range(KK):
                iv = idx_v[jn, pl.ds(g * 16, 16)]
                labs.append(plsc.load_gather(tgt_v, [iv]))
            best = jnp.full((16,), -1, jnp.int32)
            for i in range(KK):
                cnt = zero
                for jn in range(KK):
                    cnt = cnt + jnp.where(
                        (nbr > jn) & (labs[jn] == labs[i]), one, zero)
                key = jnp.where(nbr > i,
                                cnt * 1024 + (1023 - labs[i]),
                                -one)
                best = jnp.maximum(best, key)
            lab = 1023 - (best & (1024 * one - one))
            pred_v[pl.ds(g * 16, 16)] = lab.astype(jnp.float32)
        pltpu.sync_copy(pred_v, out_hbm.at[pl.ds(base, qw)])

    return vote(idx_t, support_target, nbr_vec)


def kernel(query_image, support_image, support_target, neighbors):
    q = query_image.reshape(query_image.shape[0], -1)
    s = support_image.reshape(support_image.shape[0], -1)
    idx = _topk_tc(q, s.T)
    idx_t = idx[:, :K].T
    nbr_vec = jnp.full((16,), neighbors, jnp.int32)
    return _vote_sc(idx_t, support_target, nbr_vec)


# speedup vs baseline: 3.2699x; 1.0007x over previous
import functools

import jax
import jax.numpy as jnp
from jax import lax
from jax.experimental import pallas as pl
from jax.experimental.pallas import tpu as pltpu
from jax.experimental.pallas import tpu_sc as plsc

K = 5
KPAD = 8
QBLK = 256
SBLK = 2048
IDX_BIG = 2**30


def _topk_body(q_ref, st_ref, out_ref, vals_ref, idxs_ref, q2_ref, s2_ref):
    i = pl.program_id(0)
    j = pl.program_id(1)
    nj = pl.num_programs(1)

    @pl.when(j == 0)
    def _init():
        vals_ref[...] = jnp.full((QBLK, KPAD), jnp.inf, jnp.float32)
        idxs_ref[...] = jnp.full((QBLK, KPAD), IDX_BIG, jnp.int32)
        q = q_ref[...]
        q2_ref[...] = jnp.sum(q * q, axis=1, keepdims=True)

    @pl.when(i == 0)
    def _s2():
        st = st_ref[...]
        s2_ref[j] = jnp.sum(st * st, axis=0, keepdims=True)

    dots = lax.dot_general(q_ref[...], st_ref[...], (((1,), (0,)), ((), ())),
                           preferred_element_type=jnp.float32)
    d2 = (q2_ref[...] + s2_ref[j]) - 2.0 * dots

    col = lax.broadcasted_iota(jnp.int32, (QBLK, SBLK), 1)
    base = j * SBLK
    vals = vals_ref[...]
    idxs = idxs_ref[...]
    for _ in range(K):
        m = jnp.min(d2, axis=1, keepdims=True)
        c = jnp.min(jnp.where(d2 == m, col, IDX_BIG), axis=1, keepdims=True)
        d2 = jnp.where(col == c, jnp.inf, d2)
        gi = c + base
        lt = (m < vals).astype(jnp.int32)
        p = KPAD - jnp.sum(lt, axis=1, keepdims=True)
        kcol = lax.broadcasted_iota(jnp.int32, (QBLK, KPAD), 1)
        sh_vals = jnp.concatenate([vals[:, :1], vals[:, : KPAD - 1]], axis=1)
        sh_idxs = jnp.concatenate([idxs[:, :1], idxs[:, : KPAD - 1]], axis=1)
        vals = jnp.where(kcol < p, vals, jnp.where(kcol == p, m, sh_vals))
        idxs = jnp.where(kcol < p, idxs, jnp.where(kcol == p, gi, sh_idxs))
    vals_ref[...] = vals
    idxs_ref[...] = idxs

    @pl.when(j == nj - 1)
    def _emit():
        out_ref[...] = idxs_ref[...].T


def _topk_tc(q, st):
    Q, D = q.shape
    S = st.shape[1]
    nj = S // SBLK
    return pl.pallas_call(
        _topk_body,
        grid=(Q // QBLK, nj),
        in_specs=[
            pl.BlockSpec((QBLK, D), lambda i, j: (i, 0)),
            pl.BlockSpec((D, SBLK), lambda i, j: (0, j)),
        ],
        out_specs=pl.BlockSpec((KPAD, QBLK), lambda i, j: (0, i)),
        out_shape=jax.ShapeDtypeStruct((KPAD, Q), jnp.int32),
        scratch_shapes=[
            pltpu.VMEM((QBLK, KPAD), jnp.float32),
            pltpu.VMEM((QBLK, KPAD), jnp.int32),
            pltpu.VMEM((QBLK, 1), jnp.float32),
            pltpu.VMEM((nj, 1, SBLK), jnp.float32),
        ],
        compiler_params=pltpu.CompilerParams(
            dimension_semantics=("arbitrary", "arbitrary"),
        ),
    )(q, st)


def _vote_sc(idx_t, support_target, nbr_vec):
    Q = idx_t.shape[1]
    S = support_target.shape[0]
    info = plsc.get_sparse_core_info()
    nw = info.num_cores * info.num_subcores
    qw = Q // nw
    ng = qw // 16

    mesh = plsc.VectorSubcoreMesh(core_axis_name="c", subcore_axis_name="s")

    @functools.partial(
        pl.kernel,
        mesh=mesh,
        out_type=jax.ShapeDtypeStruct((Q,), jnp.float32),
        scratch_types=[
            pltpu.VMEM((K, qw), jnp.int32),
            pltpu.VMEM((S,), jnp.int32),
            pltpu.VMEM((16,), jnp.int32),
            pltpu.VMEM((qw,), jnp.float32),
        ],
        compiler_params=pltpu.CompilerParams(needs_layout_passes=False),
    )
    def vote(idx_hbm, tgt_hbm, nbr_hbm, out_hbm, idx_v, tgt_v, nbr_v, pred_v):
        wid = lax.axis_index("s") * info.num_cores + lax.axis_index("c")
        base = wid * qw
        pltpu.sync_copy(tgt_hbm, tgt_v)
        pltpu.sync_copy(idx_hbm.at[pl.ds(0, K), pl.ds(base, qw)], idx_v)
        pltpu.sync_copy(nbr_hbm, nbr_v)
        nbr = nbr_v[...]
        one = jnp.ones((16,), jnp.int32)
        zero = jnp.zeros((16,), jnp.int32)
        for g in range(ng):
            labs = []
            for jn in range(K):
                iv = idx_v[jn, pl.ds(g * 16, 16)]
                labs.append(plsc.load_gather(tgt_v, [iv]))
            best = jnp.full((16,), -1, jnp.int32)
            for i in range(K):
                cnt = zero
                for jn in range(K):
                    cnt = cnt + jnp.where(
                        (nbr > jn) & (labs[jn] == labs[i]), one, zero)
                key = jnp.where(nbr > i,
                                cnt * 1024 + (1023 - labs[i]),
                                -one)
                best = jnp.maximum(best, key)
            lab = 1023 - (best & (1024 * one - one))
            pred_v[pl.ds(g * 16, 16)] = lab.astype(jnp.float32)
        pltpu.sync_copy(pred_v, out_hbm.at[pl.ds(base, qw)])

    return vote(idx_t, support_target, nbr_vec)


def kernel(query_image, support_image, support_target, neighbors):
    q = query_image.reshape(query_image.shape[0], -1)
    s = support_image.reshape(support_image.shape[0], -1)
    idx_t = _topk_tc(q, s.T)
    nbr_vec = jnp.full((16,), neighbors, jnp.int32)
    return _vote_sc(idx_t, support_target, nbr_vec)
